# Initial kernel scaffold; baseline (speedup 1.0000x reference)
import jax
import jax.numpy as jnp
from jax.experimental import pallas as pl


def kernel(x, table):
    raise NotImplementedError("write your pallas kernel here")


# speedup vs baseline: 3.4046x; 3.4046x over previous
import functools

import jax
import jax.numpy as jnp
from jax import lax
from jax.experimental import pallas as pl
from jax.experimental.pallas import tpu as pltpu
from jax.experimental.pallas import tpu_sc as plsc

EMBED = 64
BATCH = 16384
NUM_CORES = 2
NUM_SUBCORES = 16
NUM_WORKERS = NUM_CORES * NUM_SUBCORES
B_PER_W = BATCH // NUM_WORKERS
CHUNK = 128
NCHUNK = B_PER_W // CHUNK


@functools.partial(
    pl.kernel,
    mesh=plsc.VectorSubcoreMesh(core_axis_name="c", subcore_axis_name="s"),
    out_type=jax.ShapeDtypeStruct((BATCH, EMBED), jnp.float32),
    compiler_params=pltpu.CompilerParams(use_tc_tiling_on_sc=False),
    scratch_types=[
        pltpu.VMEM((B_PER_W,), jnp.int32),
        pltpu.VMEM((B_PER_W, EMBED), jnp.float32),
        pltpu.SemaphoreType.DMA,
    ],
)
def _gather_kernel(idx_hbm, table_hbm, out_hbm, idx_v, rows_v, sem):
    wid = lax.axis_index("s") * NUM_CORES + lax.axis_index("c")
    base = wid * B_PER_W
    pltpu.sync_copy(idx_hbm.at[pl.ds(base, B_PER_W)], idx_v)
    copies = []
    for j in range(NCHUNK):
        cp = pltpu.async_copy(
            table_hbm.at[idx_v.at[pl.ds(j * CHUNK, CHUNK)]],
            rows_v.at[pl.ds(j * CHUNK, CHUNK)],
            sem,
        )
        copies.append(cp)
    for cp in copies:
        cp.wait()
    pltpu.sync_copy(rows_v, out_hbm.at[pl.ds(base, B_PER_W)])


def kernel(x, table):
    idx = x[:, -1].astype(jnp.int32)
    return _gather_kernel(idx, table)


# speedup vs baseline: 5.1704x; 1.5186x over previous
import functools

import jax
import jax.numpy as jnp
from jax import lax
from jax.experimental import pallas as pl
from jax.experimental.pallas import tpu as pltpu
from jax.experimental.pallas import tpu_sc as plsc

EMBED = 64
BATCH = 16384
NUM_CORES = 2
NUM_SUBCORES = 16
NUM_WORKERS = NUM_CORES * NUM_SUBCORES
B_PER_W = BATCH // NUM_WORKERS
L = 16
NGRP = B_PER_W // L


@functools.partial(
    pl.kernel,
    mesh=plsc.VectorSubcoreMesh(core_axis_name="c", subcore_axis_name="s"),
    out_type=jax.ShapeDtypeStruct((BATCH, EMBED), jnp.float32),
    scratch_types=[
        pltpu.VMEM((B_PER_W,), jnp.int32),
        pltpu.SMEM((B_PER_W,), jnp.int32),
        pltpu.VMEM((L, 8, EMBED), jnp.float32),
        pltpu.VMEM((B_PER_W, EMBED), jnp.float32),
        pltpu.SemaphoreType.DMA,
    ],
)
def _tile_gather(idx_hbm, tab_hbm, outT_hbm, idx_v, idx_s, tiles_v, stg_v, sem):
    wid = lax.axis_index("s") * NUM_CORES + lax.axis_index("c")
    base = wid * B_PER_W
    pltpu.sync_copy(idx_hbm.at[pl.ds(base, B_PER_W)], idx_v)
    lane = lax.iota(jnp.int32, L)

    def group(g, carry):
        vec = idx_v[pl.ds(g * L, L)]
        vs = []
        for j in range(L):
            vj = vec[j]
            vs.append(vj)
            vb = pl.multiple_of((vj >> 3) << 3, 8)
            pltpu.async_copy(
                tab_hbm.at[pl.ds(vb, 8), :], tiles_v.at[j], sem
            )
        for j in range(L):
            pltpu.make_async_copy(
                tab_hbm.at[pl.ds(0, 8), :], tiles_v.at[j], sem
            ).wait()
        for j in range(L):
            r = vs[j] & 7
            col = g * L + j
            for k in range(EMBED // L):
                stg_v[col, pl.ds(k * L, L)] = tiles_v[j, r, pl.ds(k * L, L)]
        return carry

    lax.fori_loop(0, NGRP, group, 0)
    pltpu.sync_copy(stg_v, outT_hbm.at[pl.ds(base, B_PER_W), :])


def kernel(x, table):
    idx = x[:, -1].astype(jnp.int32)
    return _tile_gather(idx, table)


# speedup vs baseline: 5.3245x; 1.0298x over previous
import functools

import jax
import jax.numpy as jnp
from jax import lax
from jax.experimental import pallas as pl
from jax.experimental.pallas import tpu as pltpu
from jax.experimental.pallas import tpu_sc as plsc

EMBED = 64
BATCH = 16384
NUM_CORES = 2
NUM_SUBCORES = 16
NUM_WORKERS = NUM_CORES * NUM_SUBCORES
B_PER_W = BATCH // NUM_WORKERS
L = 16
NGRP = B_PER_W // L


@functools.partial(
    pl.kernel,
    mesh=plsc.VectorSubcoreMesh(core_axis_name="c", subcore_axis_name="s"),
    out_type=jax.ShapeDtypeStruct((BATCH, EMBED), jnp.float32),
    scratch_types=[
        pltpu.VMEM((B_PER_W,), jnp.int32),
        pltpu.VMEM((L, 8, EMBED), jnp.float32),
        pltpu.VMEM((L, 8, EMBED), jnp.float32),
        pltpu.VMEM((B_PER_W, EMBED), jnp.float32),
        pltpu.SemaphoreType.DMA,
        pltpu.SemaphoreType.DMA,
    ],
)
def _tile_gather(idx_hbm, tab_hbm, out_hbm, idx_v, buf_a, buf_b, stg_v,
                 sem_a, sem_b):
    wid = lax.axis_index("s") * NUM_CORES + lax.axis_index("c")
    base = wid * B_PER_W
    pltpu.sync_copy(idx_hbm.at[pl.ds(base, B_PER_W)], idx_v)

    def fire(g, buf, sem):
        vec = idx_v[pl.ds(g * L, L)]
        vs = []
        for j in range(L):
            vj = vec[j]
            vs.append(vj)
            vb = pl.multiple_of((vj >> 3) << 3, 8)
            pltpu.async_copy(tab_hbm.at[pl.ds(vb, 8), :], buf.at[j], sem)
        return vs

    def drain_extract(g, vs, buf, sem):
        for j in range(L):
            pltpu.make_async_copy(
                tab_hbm.at[pl.ds(0, 8), :], buf.at[j], sem
            ).wait()
        for j in range(L):
            r = vs[j] & 7
            col = g * L + j
            for k in range(EMBED // L):
                stg_v[col, pl.ds(k * L, L)] = buf[j, r, pl.ds(k * L, L)]

    def pair(i, carry):
        g = i * 2
        vs_a = fire(g, buf_a, sem_a)
        vs_b = fire(g + 1, buf_b, sem_b)
        drain_extract(g, vs_a, buf_a, sem_a)
        drain_extract(g + 1, vs_b, buf_b, sem_b)
        return carry

    lax.fori_loop(0, NGRP // 2, pair, 0)
    pltpu.sync_copy(stg_v, out_hbm.at[pl.ds(base, B_PER_W), :])


def kernel(x, table):
    idx = x[:, -1].astype(jnp.int32)
    return _tile_gather(idx, table)


# speedup vs baseline: 5.4748x; 1.0282x over previous
import functools

import jax
import jax.numpy as jnp
from jax import lax
from jax.experimental import pallas as pl
from jax.experimental.pallas import tpu as pltpu
from jax.experimental.pallas import tpu_sc as plsc

EMBED = 64
BATCH = 16384
NUM_CORES = 2
NUM_SUBCORES = 16
NUM_WORKERS = NUM_CORES * NUM_SUBCORES
B_PER_W = BATCH // NUM_WORKERS
L = 16
G = 32
NGRP = B_PER_W // G


@functools.partial(
    pl.kernel,
    mesh=plsc.VectorSubcoreMesh(core_axis_name="c", subcore_axis_name="s"),
    out_type=jax.ShapeDtypeStruct((BATCH, EMBED), jnp.float32),
    scratch_types=[
        pltpu.VMEM((B_PER_W,), jnp.int32),
        pltpu.VMEM((G, 8, EMBED), jnp.float32),
        pltpu.VMEM((G, 8, EMBED), jnp.float32),
        pltpu.VMEM((G, EMBED), jnp.float32),
        pltpu.VMEM((G, EMBED), jnp.float32),
        pltpu.SemaphoreType.DMA,
        pltpu.SemaphoreType.DMA,
        pltpu.SemaphoreType.DMA,
        pltpu.SemaphoreType.DMA,
    ],
)
def _tile_gather(idx_hbm, tab_hbm, out_hbm, idx_v, buf_a, buf_b, row_a,
                 row_b, sem_a, sem_b, sem_oa, sem_ob):
    wid = lax.axis_index("s") * NUM_CORES + lax.axis_index("c")
    base = wid * B_PER_W
    pltpu.sync_copy(idx_hbm.at[pl.ds(base, B_PER_W)], idx_v)

    def fire(g, buf, sem):
        for h in range(G // L):
            vec = idx_v[pl.ds(g * G + h * L, L)]
            for j in range(L):
                vb = pl.multiple_of((vec[j] >> 3) << 3, 8)
                pltpu.async_copy(
                    tab_hbm.at[pl.ds(vb, 8), :], buf.at[h * L + j], sem
                )

    def out_slab(g):
        return out_hbm.at[pl.ds(base + g * G, G), :]

    def drain_extract(g, buf, sem, row, sem_o):
        for j in range(G):
            pltpu.make_async_copy(
                tab_hbm.at[pl.ds(0, 8), :], buf.at[j], sem
            ).wait()

        @pl.when(g >= 2)
        def _():
            pltpu.make_async_copy(row, out_slab(g - 2), sem_o).wait()

        for h in range(G // L):
            vec = idx_v[pl.ds(g * G + h * L, L)]
            for j in range(L):
                r = vec[j] & 7
                for k in range(EMBED // L):
                    row[h * L + j, pl.ds(k * L, L)] = buf[h * L + j, r,
                                                          pl.ds(k * L, L)]
        pltpu.async_copy(row, out_slab(g), sem_o)

    fire(0, buf_a, sem_a)
    fire(1, buf_b, sem_b)

    def pair(i, carry):
        g = i * 2
        drain_extract(g, buf_a, sem_a, row_a, sem_oa)

        @pl.when(g + 2 < NGRP)
        def _():
            fire(g + 2, buf_a, sem_a)

        drain_extract(g + 1, buf_b, sem_b, row_b, sem_ob)

        @pl.when(g + 3 < NGRP)
        def _():
            fire(g + 3, buf_b, sem_b)

        return carry

    lax.fori_loop(0, NGRP // 2, pair, 0)
    pltpu.make_async_copy(row_a, out_slab(NGRP - 2), sem_oa).wait()
    pltpu.make_async_copy(row_b, out_slab(NGRP - 1), sem_ob).wait()


def kernel(x, table):
    idx = x[:, -1].astype(jnp.int32)
    return _tile_gather(idx, table)


# speedup vs baseline: 7.9224x; 1.4471x over previous
import functools

import jax
import jax.numpy as jnp
from jax import lax
from jax.experimental import pallas as pl
from jax.experimental.pallas import tpu as pltpu
from jax.experimental.pallas import tpu_sc as plsc

EMBED = 64
BATCH = 16384
NUM_CORES = 2
NUM_SUBCORES = 16
NUM_WORKERS = NUM_CORES * NUM_SUBCORES
B_PER_W = BATCH // NUM_WORKERS
L = 16
G = 32
NGRP = B_PER_W // G


@functools.partial(
    pl.kernel,
    mesh=plsc.VectorSubcoreMesh(core_axis_name="c", subcore_axis_name="s"),
    out_type=jax.ShapeDtypeStruct((BATCH, EMBED), jnp.float32),
    scratch_types=[
        pltpu.VMEM((B_PER_W,), jnp.int32),
        pltpu.VMEM((G, 8, EMBED), jnp.float32),
        pltpu.VMEM((G, 8, EMBED), jnp.float32),
        pltpu.VMEM((G, EMBED), jnp.float32),
        pltpu.VMEM((G, EMBED), jnp.float32),
        pltpu.SemaphoreType.DMA,
        pltpu.SemaphoreType.DMA,
        pltpu.SemaphoreType.DMA,
        pltpu.SemaphoreType.DMA,
    ],
)
def _tile_gather(idx_hbm, tab_hbm, out_hbm, idx_v, buf_a, buf_b, row_a,
                 row_b, sem_a, sem_b, sem_oa, sem_ob):
    wid = lax.axis_index("s") * NUM_CORES + lax.axis_index("c")
    base = wid * B_PER_W
    pltpu.sync_copy(idx_hbm.at[pl.ds(base, B_PER_W)], idx_v)

    def fire(g, buf, sem):
        for h in range(G // L):
            vec = idx_v[pl.ds(g * G + h * L, L)]
            for j in range(L):
                q = vec[j] >> 3
                pltpu.async_copy(tab_hbm.at[q], buf.at[h * L + j], sem)

    def out_slab(g):
        return out_hbm.at[pl.ds(base + g * G, G), :]

    def drain_extract(g, buf, sem, row, sem_o):
        for j in range(G):
            pltpu.make_async_copy(tab_hbm.at[0], buf.at[j], sem).wait()

        @pl.when(g >= 2)
        def _():
            pltpu.make_async_copy(row, out_slab(g - 2), sem_o).wait()

        for h in range(G // L):
            vec = idx_v[pl.ds(g * G + h * L, L)]
            for j in range(L):
                r = vec[j] & 7
                for k in range(EMBED // L):
                    row[h * L + j, pl.ds(k * L, L)] = buf[h * L + j, r,
                                                          pl.ds(k * L, L)]
        pltpu.async_copy(row, out_slab(g), sem_o)

    fire(0, buf_a, sem_a)
    fire(1, buf_b, sem_b)

    def pair(i, carry):
        g = i * 2
        drain_extract(g, buf_a, sem_a, row_a, sem_oa)

        @pl.when(g + 2 < NGRP)
        def _():
            fire(g + 2, buf_a, sem_a)

        drain_extract(g + 1, buf_b, sem_b, row_b, sem_ob)

        @pl.when(g + 3 < NGRP)
        def _():
            fire(g + 3, buf_b, sem_b)

        return carry

    lax.fori_loop(0, NGRP // 2, pair, 0)
    pltpu.make_async_copy(row_a, out_slab(NGRP - 2), sem_oa).wait()
    pltpu.make_async_copy(row_b, out_slab(NGRP - 1), sem_ob).wait()


def kernel(x, table):
    idx = x[:, -1].astype(jnp.int32)
    return _tile_gather(idx, table.reshape(125000, 8, EMBED))
